# baseline (speedup 1.0000x reference)
import jax
import jax.numpy as jnp
from jax import lax
from jax.experimental import pallas as pl
from jax.experimental.pallas import tpu as pltpu
from jax.experimental.pallas import tpu_sc as plsc

M = 100000
D = 64
B = 16384
SR = M // 2
SD = 2 * D

_info = plsc.get_sparse_core_info()
NC = _info.num_cores
NS = _info.num_subcores
L = _info.num_lanes
NW = NC * NS

RPW0 = 3136
RPW1 = (M - RPW0 * NS) // NS
WPAD = RPW0
NSV = WPAD // 2 // L

CHUNK = 160
LCAP = WPAD // 2 + CHUNK
IDCH = 2048
_NIDCH = B // IDCH
_NVEC = IDCH // L

CCH = 112
NCC = 14
NB = 3
CPS = CCH * NCC
CHALF = CPS * NS


def _body(buf_hbm, x_hbm, ids_hbm, mom_hbm, out_hbm,
          wtab, idsbuf,
          lidF, lieF, lioF, lidE, lieE, lidO, lioO,
          cid, cia, cib, rows, xa, xb, momv,
          cbuf0, cbuf1, cbuf2, sem_ci, sem_co, sem_a, sem_b, sem_g):
    cidx = lax.axis_index("c")
    sidx = lax.axis_index("s")
    w = cidx * NS + sidx
    lo = jnp.where(w < NS, w * RPW0, RPW0 * NS + (w - NS) * RPW1)
    rpw = jnp.where(w < NS, RPW0, RPW1)
    iota = lax.iota(jnp.int32, L)
    neg1 = iota * 0 - 1

    clo = jnp.where(w < NS, sidx * CPS,
                    CHALF + jnp.minimum(sidx * CPS, SR - CHALF - CPS))
    cbufs = [cbuf0, cbuf1, cbuf2]
    cins = [None] * NCC
    couts = [None] * NCC

    def _cin(g):
        return pltpu.async_copy(buf_hbm.at[pl.ds(clo + g * CCH, CCH)],
                                cbufs[g % NB], sem_ci.at[g % NB])

    cins[0] = _cin(0)
    cins[1] = _cin(1)

    def _ring_stage(g):
        cins[g].wait()
        couts[g] = pltpu.async_copy(cbufs[g % NB],
                                    out_hbm.at[pl.ds(clo + g * CCH, CCH)],
                                    sem_co.at[g % NB])
        if g + 2 < NCC:
            if g >= 1:
                couts[g - 1].wait()
            cins[g + 2] = _cin(g + 2)

    pltpu.sync_copy(mom_hbm, momv)
    mv = momv[...]
    omv = 1.0 - mv

    def init_body(v, carry):
        wtab[pl.ds(v * L, L)] = neg1
        return carry
    lax.fori_loop(0, WPAD // L, init_body, 0)

    for cb in range(_NIDCH):
        pltpu.sync_copy(ids_hbm.at[pl.ds(cb * IDCH, IDCH)], idsbuf)

        def vec_body(v, c2, cb=cb):
            idv = idsbuf[pl.ds(v * L, L)]
            iv = cb * IDCH + v * L + iota
            inr = (idv >= lo) & (idv < lo + rpw)
            slot = jnp.where(inr, idv - lo, 0)
            plsc.store_scatter(wtab, [slot], iv, mask=inr)
            g = plsc.load_gather(wtab, [slot], mask=inr)
            fix = inr & (iv > g)
            plsc.store_scatter(wtab, [slot], iv, mask=fix)
            return c2
        with jax.named_scope("scan_ids"):
            lax.fori_loop(0, _NVEC, vec_body, 0)
        for g in (2 * cb, 2 * cb + 1):
            if g < NCC:
                _ring_stage(g)

    def comp_body(sv, offs):
        offF, offE, offO = offs
        ls2 = (sv * L + iota) * 2
        we = plsc.load_gather(wtab, [ls2])
        wo = plsc.load_gather(wtab, [ls2 + 1])
        be = we >= 0
        bo = wo >= 0
        sid = (lo // 2) + sv * L + iota
        mF = be & bo
        mE = be & (~bo)
        mO = (~be) & bo
        plsc.store_compressed(lidF.at[pl.ds(offF, L)], sid, mask=mF)
        plsc.store_compressed(lieF.at[pl.ds(offF, L)], we, mask=mF)
        plsc.store_compressed(lioF.at[pl.ds(offF, L)], wo, mask=mF)
        plsc.store_compressed(lidE.at[pl.ds(offE, L)], sid, mask=mE)
        plsc.store_compressed(lieE.at[pl.ds(offE, L)], we, mask=mE)
        plsc.store_compressed(lidO.at[pl.ds(offO, L)], sid, mask=mO)
        plsc.store_compressed(lioO.at[pl.ds(offO, L)], wo, mask=mO)
        return (offF + jnp.sum(mF.astype(jnp.int32), axis=0),
                offE + jnp.sum(mE.astype(jnp.int32), axis=0),
                offO + jnp.sum(mO.astype(jnp.int32), axis=0))
    with jax.named_scope("compact"):
        kF, kE, kO = lax.fori_loop(0, NSV, comp_body, (0, 0, 0))

    zeros = iota * 0

    def _pad(lists, k):
        pads = [plsc.load_gather(ref, [zeros]) for ref in lists]

        def pad_body(p, carry):
            for ref, pv in zip(lists, pads):
                ref[pl.ds(k + p * L, L)] = pv
            return carry
        lax.fori_loop(0, CHUNK // L, pad_body, 0)

    _pad([lidF, lieF, lioF], kF)
    _pad([lidE, lieE], kE)
    _pad([lidO, lioO], kO)

    couts[NCC - 2].wait()
    couts[NCC - 1].wait()
    plsc.subcore_barrier()

    def _updates(lid_, lia_, lib_, k, crange, scope):
        nch = (k + CHUNK - 1) // CHUNK

        def upd_body(ci, carry):
            off = ci * CHUNK

            def cpidx_body(v, c2):
                cid[pl.ds(v * L, L)] = lid_[pl.ds(off + v * L, L)]
                cia[pl.ds(v * L, L)] = lia_[pl.ds(off + v * L, L)]
                if lib_ is not None:
                    cib[pl.ds(v * L, L)] = lib_[pl.ds(off + v * L, L)]
                return c2
            lax.fori_loop(0, CHUNK // L, cpidx_body, 0)
            ga = pltpu.async_copy(buf_hbm.at[cid], rows, sem_a)
            gb = pltpu.async_copy(x_hbm.at[cia], xa, sem_b)
            gc = None
            if lib_ is not None:
                gc = pltpu.async_copy(x_hbm.at[cib], xb, sem_g)
            ga.wait()
            gb.wait()
            if gc is not None:
                gc.wait()

            def row_body(r, c2):
                for c in crange:
                    src = xa if (lib_ is None or c < SD // L // 2) else xb
                    rv = rows[r, pl.ds(c * L, L)]
                    xv = src[r, pl.ds(c * L, L)]
                    rows[r, pl.ds(c * L, L)] = rv * mv + xv * omv
                return c2
            lax.fori_loop(0, CHUNK, row_body, 0)
            pltpu.async_copy(rows, out_hbm.at[cid], sem_a).wait()
            return carry
        with jax.named_scope(scope):
            lax.fori_loop(0, nch, upd_body, 0)

    half = SD // L // 2
    _updates(lidE, lieE, None, kE, range(0, half), "upd_even")
    _updates(lidO, lioO, None, kO, range(half, 2 * half), "upd_odd")
    _updates(lidF, lieF, lioF, kF, range(0, 2 * half), "upd_full")


@jax.jit
def kernel(buffer, x, ids, mom):
    ids32 = ids.astype(jnp.int32)
    momv = jnp.broadcast_to(mom.astype(jnp.float32), (L,))
    buf2 = jnp.concatenate([buffer[0::2], buffer[1::2]], axis=1)
    xdup = jnp.concatenate([x, x], axis=1)
    mesh = plsc.VectorSubcoreMesh(core_axis_name="c", subcore_axis_name="s")
    f = pl.kernel(
        _body,
        out_type=jax.ShapeDtypeStruct((SR, SD), jnp.float32),
        mesh=mesh,
        compiler_params=pltpu.CompilerParams(needs_layout_passes=False),
        scratch_types=[
            pltpu.VMEM((WPAD,), jnp.int32),
            pltpu.VMEM((IDCH,), jnp.int32),
            pltpu.VMEM((LCAP,), jnp.int32),
            pltpu.VMEM((LCAP,), jnp.int32),
            pltpu.VMEM((LCAP,), jnp.int32),
            pltpu.VMEM((LCAP,), jnp.int32),
            pltpu.VMEM((LCAP,), jnp.int32),
            pltpu.VMEM((LCAP,), jnp.int32),
            pltpu.VMEM((LCAP,), jnp.int32),
            pltpu.VMEM((CHUNK,), jnp.int32),
            pltpu.VMEM((CHUNK,), jnp.int32),
            pltpu.VMEM((CHUNK,), jnp.int32),
            pltpu.VMEM((CHUNK, SD), jnp.float32),
            pltpu.VMEM((CHUNK, SD), jnp.float32),
            pltpu.VMEM((CHUNK, SD), jnp.float32),
            pltpu.VMEM((L,), jnp.float32),
            pltpu.VMEM((CCH, SD), jnp.float32),
            pltpu.VMEM((CCH, SD), jnp.float32),
            pltpu.VMEM((CCH, SD), jnp.float32),
            pltpu.SemaphoreType.DMA((NB,)),
            pltpu.SemaphoreType.DMA((NB,)),
            pltpu.SemaphoreType.DMA,
            pltpu.SemaphoreType.DMA,
            pltpu.SemaphoreType.DMA,
        ],
    )
    out2 = f(buf2, xdup, ids32, momv)
    return jnp.stack([out2[:, :D], out2[:, D:]], axis=1).reshape(M, D)


# speedup vs baseline: 4.9155x; 4.9155x over previous
import jax
import jax.numpy as jnp
from jax import lax
from jax.experimental import pallas as pl
from jax.experimental.pallas import tpu as pltpu
from jax.experimental.pallas import tpu_sc as plsc

M = 100000
D = 64
B = 16384
SR = M // 2
SD = 2 * D

_info = plsc.get_sparse_core_info()
NC = _info.num_cores
NS = _info.num_subcores
L = _info.num_lanes
NW = NC * NS

RPW0 = 3136
RPW1 = (M - RPW0 * NS) // NS
WPAD = RPW0
NSV = WPAD // 2 // L

CHUNK = 160
LCAP = WPAD // 2 + CHUNK
IDCH = 2048
_NIDCH = B // IDCH
_NVEC = IDCH // L

CCH = 112
NCC = 14
NB = 3
CPS = CCH * NCC
CHALF = CPS * NS


def _body(buf_hbm, x_hbm, ids_hbm, mom_hbm, out_hbm,
          wtab, idsbuf,
          lidF, lieF, lioF, lidE, lieE, lidO, lioO,
          cid, cia, cib, rows, xa, xb, momv,
          cbuf0, cbuf1, cbuf2, sem_ci, sem_co, sem_a, sem_b, sem_g):
    cidx = lax.axis_index("c")
    sidx = lax.axis_index("s")
    w = cidx * NS + sidx
    lo = jnp.where(w < NS, w * RPW0, RPW0 * NS + (w - NS) * RPW1)
    rpw = jnp.where(w < NS, RPW0, RPW1)
    iota = lax.iota(jnp.int32, L)
    neg1 = iota * 0 - 1

    clo = jnp.where(w < NS, sidx * CPS,
                    CHALF + jnp.minimum(sidx * CPS, SR - CHALF - CPS))
    cbufs = [cbuf0, cbuf1, cbuf2]
    cins = [None] * NCC
    couts = [None] * NCC

    def _cin(g):
        return pltpu.async_copy(buf_hbm.at[pl.ds(clo + g * CCH, CCH)],
                                cbufs[g % NB], sem_ci.at[g % NB])

    cins[0] = _cin(0)
    cins[1] = _cin(1)

    def _ring_stage(g):
        cins[g].wait()
        couts[g] = pltpu.async_copy(cbufs[g % NB],
                                    out_hbm.at[pl.ds(clo + g * CCH, CCH)],
                                    sem_co.at[g % NB])
        if g + 2 < NCC:
            if g >= 1:
                couts[g - 1].wait()
            cins[g + 2] = _cin(g + 2)

    pltpu.sync_copy(mom_hbm, momv)
    mv = momv[...]
    omv = 1.0 - mv

    def init_body(v, carry):
        wtab[pl.ds(v * L, L)] = neg1
        return carry
    lax.fori_loop(0, WPAD // L, init_body, 0)

    for cb in range(_NIDCH):
        pltpu.sync_copy(ids_hbm.at[pl.ds(cb * IDCH, IDCH)], idsbuf)

        def vec_body(v, c2, cb=cb):
            idv = idsbuf[pl.ds(v * L, L)]
            iv = cb * IDCH + v * L + iota
            inr = (idv >= lo) & (idv < lo + rpw)
            slot = jnp.where(inr, idv - lo, 0)
            plsc.store_scatter(wtab, [slot], iv, mask=inr)
            g = plsc.load_gather(wtab, [slot], mask=inr)
            fix = inr & (iv > g)
            plsc.store_scatter(wtab, [slot], iv, mask=fix)
            return c2
        with jax.named_scope("scan_ids"):
            lax.fori_loop(0, _NVEC, vec_body, 0)
        for g in (2 * cb, 2 * cb + 1):
            if g < NCC:
                _ring_stage(g)

    def comp_body(sv, offs):
        offF, offE, offO = offs
        ls2 = (sv * L + iota) * 2
        we = plsc.load_gather(wtab, [ls2])
        wo = plsc.load_gather(wtab, [ls2 + 1])
        be = we >= 0
        bo = wo >= 0
        sid = (lo // 2) + sv * L + iota
        mF = be & bo
        mE = be & (~bo)
        mO = (~be) & bo
        plsc.store_compressed(lidF.at[pl.ds(offF, L)], sid, mask=mF)
        plsc.store_compressed(lieF.at[pl.ds(offF, L)], we, mask=mF)
        plsc.store_compressed(lioF.at[pl.ds(offF, L)], wo, mask=mF)
        plsc.store_compressed(lidE.at[pl.ds(offE, L)], sid, mask=mE)
        plsc.store_compressed(lieE.at[pl.ds(offE, L)], we, mask=mE)
        plsc.store_compressed(lidO.at[pl.ds(offO, L)], sid, mask=mO)
        plsc.store_compressed(lioO.at[pl.ds(offO, L)], wo, mask=mO)
        return (offF + jnp.sum(mF.astype(jnp.int32), axis=0),
                offE + jnp.sum(mE.astype(jnp.int32), axis=0),
                offO + jnp.sum(mO.astype(jnp.int32), axis=0))
    with jax.named_scope("compact"):
        kF, kE, kO = lax.fori_loop(0, NSV, comp_body, (0, 0, 0))

    zeros = iota * 0

    def _pad(lists, k):
        pads = [plsc.load_gather(ref, [zeros]) for ref in lists]

        def pad_body(p, carry):
            for ref, pv in zip(lists, pads):
                ref[pl.ds(k + p * L, L)] = pv
            return carry
        lax.fori_loop(0, CHUNK // L, pad_body, 0)

    _pad([lidF, lieF, lioF], kF)
    _pad([lidE, lieE], kE)
    _pad([lidO, lioO], kO)

    couts[NCC - 2].wait()
    couts[NCC - 1].wait()
    plsc.subcore_barrier()

    def _updates(lid_, lia_, lib_, k, crange, scope):
        nch = (k + CHUNK - 1) // CHUNK

        def upd_body(ci, carry):
            off = ci * CHUNK

            def cpidx_body(v, c2):
                cid[pl.ds(v * L, L)] = lid_[pl.ds(off + v * L, L)]
                cia[pl.ds(v * L, L)] = lia_[pl.ds(off + v * L, L)]
                if lib_ is not None:
                    cib[pl.ds(v * L, L)] = lib_[pl.ds(off + v * L, L)]
                return c2
            lax.fori_loop(0, CHUNK // L, cpidx_body, 0)
            ga = pltpu.async_copy(buf_hbm.at[cid], rows, sem_a)
            gb = pltpu.async_copy(x_hbm.at[cia], xa, sem_b)
            gc = None
            if lib_ is not None:
                gc = pltpu.async_copy(x_hbm.at[cib], xb, sem_g)
            ga.wait()
            gb.wait()
            if gc is not None:
                gc.wait()

            def row_body(r, c2):
                for c in crange:
                    src = xa if (lib_ is None or c < SD // L // 2) else xb
                    rv = rows[r, pl.ds(c * L, L)]
                    xv = src[r, pl.ds(c * L, L)]
                    rows[r, pl.ds(c * L, L)] = rv * mv + xv * omv
                return c2
            lax.fori_loop(0, CHUNK, row_body, 0)
            pltpu.async_copy(rows, out_hbm.at[cid], sem_a).wait()
            return carry
        with jax.named_scope(scope):
            lax.fori_loop(0, nch, upd_body, 0)

    half = SD // L // 2
    _updates(lidE, lieE, None, kE, range(0, half), "upd_even")
    _updates(lidO, lioO, None, kO, range(half, 2 * half), "upd_odd")
    _updates(lidF, lieF, lioF, kF, range(0, 2 * half), "upd_full")


@jax.jit
def kernel(buffer, x, ids, mom):
    ids32 = ids.astype(jnp.int32)
    momv = jnp.broadcast_to(mom.astype(jnp.float32), (L,))
    buf2 = jnp.reshape(buffer, (SR, SD))
    xdup = jnp.concatenate([x, x], axis=1)
    mesh = plsc.VectorSubcoreMesh(core_axis_name="c", subcore_axis_name="s")
    f = pl.kernel(
        _body,
        out_type=jax.ShapeDtypeStruct((SR, SD), jnp.float32),
        mesh=mesh,
        compiler_params=pltpu.CompilerParams(needs_layout_passes=False),
        scratch_types=[
            pltpu.VMEM((WPAD,), jnp.int32),
            pltpu.VMEM((IDCH,), jnp.int32),
            pltpu.VMEM((LCAP,), jnp.int32),
            pltpu.VMEM((LCAP,), jnp.int32),
            pltpu.VMEM((LCAP,), jnp.int32),
            pltpu.VMEM((LCAP,), jnp.int32),
            pltpu.VMEM((LCAP,), jnp.int32),
            pltpu.VMEM((LCAP,), jnp.int32),
            pltpu.VMEM((LCAP,), jnp.int32),
            pltpu.VMEM((CHUNK,), jnp.int32),
            pltpu.VMEM((CHUNK,), jnp.int32),
            pltpu.VMEM((CHUNK,), jnp.int32),
            pltpu.VMEM((CHUNK, SD), jnp.float32),
            pltpu.VMEM((CHUNK, SD), jnp.float32),
            pltpu.VMEM((CHUNK, SD), jnp.float32),
            pltpu.VMEM((L,), jnp.float32),
            pltpu.VMEM((CCH, SD), jnp.float32),
            pltpu.VMEM((CCH, SD), jnp.float32),
            pltpu.VMEM((CCH, SD), jnp.float32),
            pltpu.SemaphoreType.DMA((NB,)),
            pltpu.SemaphoreType.DMA((NB,)),
            pltpu.SemaphoreType.DMA,
            pltpu.SemaphoreType.DMA,
            pltpu.SemaphoreType.DMA,
        ],
    )
    out2 = f(buf2, xdup, ids32, momv)
    return jnp.reshape(out2, (M, D))


# speedup vs baseline: 5.1541x; 1.0485x over previous
import jax
import jax.numpy as jnp
from jax import lax
from jax.experimental import pallas as pl
from jax.experimental.pallas import tpu as pltpu
from jax.experimental.pallas import tpu_sc as plsc

M = 100000
D = 64
B = 16384
SR = M // 2
SD = 2 * D

_info = plsc.get_sparse_core_info()
NC = _info.num_cores
NS = _info.num_subcores
L = _info.num_lanes
NW = NC * NS

RPW0 = 3136
RPW1 = (M - RPW0 * NS) // NS
WPAD = RPW0
NSV = WPAD // 2 // L

CHUNK = 160
LCAP = WPAD // 2 + CHUNK
IDCH = 2048
_NIDCH = B // IDCH
_NVEC = IDCH // L

CCH = 112
NCC = 14
NB = 3
CPS = CCH * NCC
CHALF = CPS * NS


def _body(buf_hbm, x_hbm, ids_hbm, mom_hbm, out_hbm,
          wtab, idsbuf0, idsbuf1,
          lidF, lieF, lioF, lidE, lieE, lidO, lioO,
          cid, cia, cib, rows, xa, xb, momv,
          cbuf0, cbuf1, cbuf2, sem_ci, sem_co, sem_a, sem_b, sem_g,
          sem_i0, sem_i1):
    cidx = lax.axis_index("c")
    sidx = lax.axis_index("s")
    w = cidx * NS + sidx
    lo = w * RPW0
    rpw = jnp.minimum(RPW0, M - lo)
    iota = lax.iota(jnp.int32, L)
    neg1 = iota * 0 - 1

    clo = jnp.where(w < NS, sidx * CPS,
                    CHALF + jnp.minimum(sidx * CPS, SR - CHALF - CPS))
    cbufs = [cbuf0, cbuf1, cbuf2]
    cins = [None] * NCC
    couts = [None] * NCC

    def _cin(g):
        return pltpu.async_copy(buf_hbm.at[pl.ds(clo + g * CCH, CCH)],
                                cbufs[g % NB], sem_ci.at[g % NB])

    cins[0] = _cin(0)
    cins[1] = _cin(1)

    def _ring_stage(g):
        cins[g].wait()
        couts[g] = pltpu.async_copy(cbufs[g % NB],
                                    out_hbm.at[pl.ds(clo + g * CCH, CCH)],
                                    sem_co.at[g % NB])
        if g + 2 < NCC:
            if g >= 1:
                couts[g - 1].wait()
            cins[g + 2] = _cin(g + 2)

    pltpu.sync_copy(mom_hbm, momv)
    mv = momv[...]
    omv = 1.0 - mv

    def init_body(v, carry):
        wtab[pl.ds(v * L, L)] = neg1
        return carry
    lax.fori_loop(0, WPAD // L, init_body, 0)

    idsbufs = [idsbuf0, idsbuf1]
    idsems = [sem_i0, sem_i1]

    def _iin(cb):
        return pltpu.async_copy(ids_hbm.at[pl.ds(cb * IDCH, IDCH)],
                                idsbufs[cb % 2], idsems[cb % 2])

    iins = [None] * _NIDCH
    iins[0] = _iin(0)
    iins[1] = _iin(1)
    for cb in range(_NIDCH):
        iins[cb].wait()
        idsbuf = idsbufs[cb % 2]

        def vec_body(v, c2, cb=cb):
            idv = idsbuf[pl.ds(v * L, L)]
            iv = cb * IDCH + v * L + iota
            inr = (idv >= lo) & (idv < lo + rpw)
            slot = jnp.where(inr, idv - lo, 0)
            plsc.store_scatter(wtab, [slot], iv, mask=inr)
            g = plsc.load_gather(wtab, [slot], mask=inr)
            fix = inr & (iv > g)
            plsc.store_scatter(wtab, [slot], iv, mask=fix)
            return c2
        with jax.named_scope("scan_ids"):
            lax.fori_loop(0, _NVEC, vec_body, 0)
        if cb + 2 < _NIDCH:
            iins[cb + 2] = _iin(cb + 2)
        for g in (2 * cb, 2 * cb + 1):
            if g < NCC:
                _ring_stage(g)

    def comp_body(sv, offs):
        offF, offE, offO = offs
        ls2 = (sv * L + iota) * 2
        we = plsc.load_gather(wtab, [ls2])
        wo = plsc.load_gather(wtab, [ls2 + 1])
        be = we >= 0
        bo = wo >= 0
        sid = (lo // 2) + sv * L + iota
        mF = be & bo
        mE = be & (~bo)
        mO = (~be) & bo
        plsc.store_compressed(lidF.at[pl.ds(offF, L)], sid, mask=mF)
        plsc.store_compressed(lieF.at[pl.ds(offF, L)], we, mask=mF)
        plsc.store_compressed(lioF.at[pl.ds(offF, L)], wo, mask=mF)
        plsc.store_compressed(lidE.at[pl.ds(offE, L)], sid, mask=mE)
        plsc.store_compressed(lieE.at[pl.ds(offE, L)], we, mask=mE)
        plsc.store_compressed(lidO.at[pl.ds(offO, L)], sid, mask=mO)
        plsc.store_compressed(lioO.at[pl.ds(offO, L)], wo, mask=mO)
        return (offF + jnp.sum(mF.astype(jnp.int32), axis=0),
                offE + jnp.sum(mE.astype(jnp.int32), axis=0),
                offO + jnp.sum(mO.astype(jnp.int32), axis=0))
    with jax.named_scope("compact"):
        kF, kE, kO = lax.fori_loop(0, NSV, comp_body, (0, 0, 0))

    zeros = iota * 0

    def _pad(lists, k):
        pads = [plsc.load_gather(ref, [zeros]) for ref in lists]

        def pad_body(p, carry):
            for ref, pv in zip(lists, pads):
                ref[pl.ds(k + p * L, L)] = pv
            return carry
        lax.fori_loop(0, CHUNK // L, pad_body, 0)

    _pad([lidF, lieF, lioF], kF)
    _pad([lidE, lieE], kE)
    _pad([lidO, lioO], kO)

    couts[NCC - 2].wait()
    couts[NCC - 1].wait()

    def _updates(lid_, lia_, lib_, k, crange, scope):
        nch = (k + CHUNK - 1) // CHUNK

        def upd_body(ci, carry):
            off = ci * CHUNK

            def cpidx_body(v, c2):
                cid[pl.ds(v * L, L)] = lid_[pl.ds(off + v * L, L)]
                cia[pl.ds(v * L, L)] = lia_[pl.ds(off + v * L, L)]
                if lib_ is not None:
                    cib[pl.ds(v * L, L)] = lib_[pl.ds(off + v * L, L)]
                return c2
            lax.fori_loop(0, CHUNK // L, cpidx_body, 0)
            ga = pltpu.async_copy(buf_hbm.at[cid], rows, sem_a)
            gb = pltpu.async_copy(x_hbm.at[cia], xa, sem_b)
            gc = None
            if lib_ is not None:
                gc = pltpu.async_copy(x_hbm.at[cib], xb, sem_g)
            ga.wait()
            gb.wait()
            if gc is not None:
                gc.wait()

            def row_body(r, c2):
                for c in crange:
                    src = xa if (lib_ is None or c < SD // L // 2) else xb
                    rv = rows[r, pl.ds(c * L, L)]
                    xv = src[r, pl.ds(c * L, L)]
                    rows[r, pl.ds(c * L, L)] = rv * mv + xv * omv
                return c2
            lax.fori_loop(0, CHUNK, row_body, 0)
            pltpu.async_copy(rows, out_hbm.at[cid], sem_a).wait()
            return carry
        with jax.named_scope(scope):
            lax.fori_loop(0, nch, upd_body, 0)

    half = SD // L // 2
    _updates(lidE, lieE, None, kE, range(0, half), "upd_even")
    _updates(lidO, lioO, None, kO, range(half, 2 * half), "upd_odd")
    _updates(lidF, lieF, lioF, kF, range(0, 2 * half), "upd_full")


@jax.jit
def kernel(buffer, x, ids, mom):
    ids32 = ids.astype(jnp.int32)
    momv = jnp.broadcast_to(mom.astype(jnp.float32), (L,))
    buf2 = jnp.reshape(buffer, (SR, SD))
    xdup = jnp.concatenate([x, x], axis=1)
    mesh = plsc.VectorSubcoreMesh(core_axis_name="c", subcore_axis_name="s")
    f = pl.kernel(
        _body,
        out_type=jax.ShapeDtypeStruct((SR, SD), jnp.float32),
        mesh=mesh,
        compiler_params=pltpu.CompilerParams(needs_layout_passes=False),
        scratch_types=[
            pltpu.VMEM((WPAD,), jnp.int32),
            pltpu.VMEM((IDCH,), jnp.int32),
            pltpu.VMEM((IDCH,), jnp.int32),
            pltpu.VMEM((LCAP,), jnp.int32),
            pltpu.VMEM((LCAP,), jnp.int32),
            pltpu.VMEM((LCAP,), jnp.int32),
            pltpu.VMEM((LCAP,), jnp.int32),
            pltpu.VMEM((LCAP,), jnp.int32),
            pltpu.VMEM((LCAP,), jnp.int32),
            pltpu.VMEM((LCAP,), jnp.int32),
            pltpu.VMEM((CHUNK,), jnp.int32),
            pltpu.VMEM((CHUNK,), jnp.int32),
            pltpu.VMEM((CHUNK,), jnp.int32),
            pltpu.VMEM((CHUNK, SD), jnp.float32),
            pltpu.VMEM((CHUNK, SD), jnp.float32),
            pltpu.VMEM((CHUNK, SD), jnp.float32),
            pltpu.VMEM((L,), jnp.float32),
            pltpu.VMEM((CCH, SD), jnp.float32),
            pltpu.VMEM((CCH, SD), jnp.float32),
            pltpu.VMEM((CCH, SD), jnp.float32),
            pltpu.SemaphoreType.DMA((NB,)),
            pltpu.SemaphoreType.DMA((NB,)),
            pltpu.SemaphoreType.DMA,
            pltpu.SemaphoreType.DMA,
            pltpu.SemaphoreType.DMA,
            pltpu.SemaphoreType.DMA,
            pltpu.SemaphoreType.DMA,
        ],
    )
    out2 = f(buf2, xdup, ids32, momv)
    return jnp.reshape(out2, (M, D))
